# baseline (speedup 1.0000x reference)
import functools

import jax
import jax.numpy as jnp
from jax import lax
from jax.experimental import pallas as pl
from jax.experimental.pallas import tpu as pltpu
from jax.experimental.pallas import tpu_sc as plsc

_B, _L, _D, _H = 16384, 200, 32, 64
_C0, _C1 = 128, 72

_info = plsc.get_sparse_core_info()
_NC, _NS = _info.num_cores, _info.num_subcores
_NW = _NC * _NS
_RPW = _B // _NW
_G = 16
_NGROUPS = _RPW // _G


def _pool_body(idx_hbm, table_hbm, out_hbm, idx_v, buf0, buf1, out_v, sem0, sem1):
    wid = lax.axis_index("s") * _NC + lax.axis_index("c")
    base = wid * _RPW

    def issue(r, buf, sem):
        tl, rr = r // 8, r % 8
        pltpu.async_copy(table_hbm.at[idx_v.at[tl, 0, rr]],
                         buf.at[pl.ds(0, _C0)], sem)
        pltpu.async_copy(table_hbm.at[idx_v.at[tl, 1, rr, pl.ds(0, _C1)]],
                         buf.at[pl.ds(_C0, _C1)], sem)

    def wait_full(buf, sem):
        pltpu.make_async_copy(table_hbm.at[pl.ds(0, _L)], buf, sem).wait()

    def acc_store(buf, r):
        a = [jnp.zeros((16,), jnp.float32) for _ in range(4)]
        b = [jnp.zeros((16,), jnp.float32) for _ in range(4)]
        for rr in range(_L):
            k = rr % 4
            a[k] = a[k] + buf[rr, pl.ds(0, 16)]
            b[k] = b[k] + buf[rr, pl.ds(16, 16)]
        out_v[r, pl.ds(0, 16)] = (a[0] + a[1]) + (a[2] + a[3])
        out_v[r, pl.ds(16, 16)] = (b[0] + b[1]) + (b[2] + b[3])

    def pair(p, carry):
        r0 = 2 * p
        issue(r0 + 1, buf1, sem1)
        wait_full(buf0, sem0)
        acc_store(buf0, r0)

        @pl.when(p + 1 < _G // 2)
        def _():
            issue(r0 + 2, buf0, sem0)

        wait_full(buf1, sem1)
        acc_store(buf1, r0 + 1)
        return carry

    def group(g, carry):
        row0 = base + g * _G
        pltpu.sync_copy(idx_hbm.at[pl.ds(row0 // 8, _G // 8)], idx_v)
        issue(0, buf0, sem0)
        lax.fori_loop(0, _G // 2, pair, 0)
        pltpu.sync_copy(out_v, out_hbm.at[pl.ds(row0, _G)])
        return carry

    lax.fori_loop(0, _NGROUPS, group, 0)


_pool = functools.partial(
    pl.kernel,
    out_type=jax.ShapeDtypeStruct((_B, _D), jnp.float32),
    mesh=plsc.VectorSubcoreMesh(core_axis_name="c", subcore_axis_name="s"),
    scratch_types=[
        pltpu.VMEM((_G // 8, 2, 8, 128), jnp.int32),
        pltpu.VMEM((_L, _D), jnp.float32),
        pltpu.VMEM((_L, _D), jnp.float32),
        pltpu.VMEM((_G, _D), jnp.float32),
        pltpu.SemaphoreType.DMA,
        pltpu.SemaphoreType.DMA,
    ],
    compiler_params=pltpu.CompilerParams(use_tc_tiling_on_sc=False),
)(_pool_body)


_BB = 2048


def _mlp_body(idx_ref, ps_ref, w1_ref, b1_ref, w2_ref, b2_ref, out_ref):
    cnt = jnp.sum((idx_ref[...] != 0).astype(jnp.float32), axis=1, keepdims=True)
    denom = jnp.maximum(cnt, 1.0)
    pooled = ps_ref[...] / denom
    h = lax.dot_general(pooled, w1_ref[...], (((1,), (1,)), ((), ())),
                        preferred_element_type=jnp.float32)
    h = jnp.maximum(h + b1_ref[...], 0.0)
    o = jnp.sum(h * w2_ref[...], axis=1, keepdims=True)
    out_ref[...] = o + b2_ref[0, 0]


_mlp = pl.pallas_call(
    _mlp_body,
    grid=(_B // _BB,),
    in_specs=[
        pl.BlockSpec((_BB, _L), lambda i: (i, 0)),
        pl.BlockSpec((_BB, _D), lambda i: (i, 0)),
        pl.BlockSpec((_H, _D), lambda i: (0, 0)),
        pl.BlockSpec((1, _H), lambda i: (0, 0)),
        pl.BlockSpec((1, _H), lambda i: (0, 0)),
        pl.BlockSpec(memory_space=pltpu.SMEM),
    ],
    out_specs=pl.BlockSpec((_BB, 1), lambda i: (i, 0)),
    out_shape=jax.ShapeDtypeStruct((_B, 1), jnp.float32),
)


def kernel(indices, table, W1, b1, W2, b2):
    idx = indices.astype(jnp.int32)
    idx_lin = jnp.pad(idx, ((0, 0), (0, 256 - _L)))
    idx_lin = idx_lin.reshape(_B // 8, 8, 2, 128).transpose(0, 2, 1, 3)
    pooled_sum = _pool(idx_lin, table)
    out = _mlp(idx, pooled_sum, W1, b1.reshape(1, _H), W2, b2.reshape(1, 1))
    return out.reshape(_B)
